# baseline (speedup 1.0000x reference)
import functools

import jax
import jax.numpy as jnp
from jax import lax
from jax.experimental import pallas as pl
from jax.experimental.pallas import tpu as pltpu
from jax.experimental.pallas import tpu_sc as plsc

_NUM_CORES = 2
_NUM_SUBCORES = 16
_LANES = 16


def _sc_gather_t(flat_table, idx, d_dim, stride):
    b = idx.shape[0]
    mesh = plsc.VectorSubcoreMesh(core_axis_name="c", subcore_axis_name="s")

    @functools.partial(
        pl.kernel,
        mesh=mesh,
        compiler_params=pltpu.CompilerParams(use_tc_tiling_on_sc=False),
        out_type=jax.ShapeDtypeStruct((d_dim * b,), jnp.float32),
        scratch_types=[
            pltpu.VMEM((b,), jnp.int32),
            pltpu.VMEM((b,), jnp.int32),
            pltpu.VMEM((b,), jnp.float32),
            pltpu.SemaphoreType.DMA,
        ],
    )
    def gather_kernel(flat_hbm, idx_hbm, out_hbm, idx_v, off_v, out_v, sem):
        d = lax.axis_index("s") * _NUM_CORES + lax.axis_index("c")
        pltpu.sync_copy(idx_hbm, idx_v)
        base = d * stride
        for s in range(b // _LANES):
            sl = pl.ds(s * _LANES, _LANES)
            off_v[sl] = idx_v[sl] + base
        pltpu.async_copy(flat_hbm.at[off_v], out_v, sem).wait()
        pltpu.sync_copy(out_v, out_hbm.at[pl.ds(d * b, b)])

    return gather_kernel(flat_table, idx)


def _tc_project_t(wt, et_flat, d, b, vt):
    v = wt.shape[1]

    def body(wt_ref, et_ref, o_ref):
        et = et_ref[...].reshape(d, b)
        o_ref[...] = lax.dot_general(
            wt_ref[...],
            et,
            (((0,), (0,)), ((), ())),
            preferred_element_type=jnp.float32,
        )

    return pl.pallas_call(
        body,
        grid=(pl.cdiv(v, vt),),
        in_specs=[
            pl.BlockSpec((d, vt), lambda j: (0, j)),
            pl.BlockSpec((d * b,), lambda j: (0,)),
        ],
        out_specs=pl.BlockSpec((vt, b), lambda j: (j, 0)),
        out_shape=jax.ShapeDtypeStruct((v, b), jnp.float32),
    )(wt, et_flat)


def kernel(x, emb_table, out_weight):
    rows, d = emb_table.shape
    b = x.shape[0]
    flat_t = emb_table.T.reshape(-1)
    et_flat = _sc_gather_t(flat_t, x.astype(jnp.int32), d, rows)
    logits_t = _tc_project_t(out_weight.T, et_flat, d, b, vt=6144)
    return logits_t.T


# speedup vs baseline: 1.0190x; 1.0190x over previous
import functools

import jax
import jax.numpy as jnp
from jax import lax
from jax.experimental import pallas as pl
from jax.experimental.pallas import tpu as pltpu
from jax.experimental.pallas import tpu_sc as plsc

_NUM_CORES = 2
_NUM_SUBCORES = 16
_LANES = 16


def _sc_gather_t(flat_table, idx, d_dim, stride):
    b = idx.shape[0]
    mesh = plsc.VectorSubcoreMesh(core_axis_name="c", subcore_axis_name="s")

    @functools.partial(
        pl.kernel,
        mesh=mesh,
        compiler_params=pltpu.CompilerParams(use_tc_tiling_on_sc=False),
        out_type=jax.ShapeDtypeStruct((d_dim * b,), jnp.float32),
        scratch_types=[
            pltpu.VMEM((b,), jnp.int32),
            pltpu.VMEM((b,), jnp.int32),
            pltpu.VMEM((b,), jnp.float32),
            pltpu.SemaphoreType.DMA,
        ],
    )
    def gather_kernel(flat_hbm, idx_hbm, out_hbm, idx_v, off_v, out_v, sem):
        d = lax.axis_index("s") * _NUM_CORES + lax.axis_index("c")
        pltpu.sync_copy(idx_hbm, idx_v)
        base = d * stride
        for s in range(b // _LANES):
            sl = pl.ds(s * _LANES, _LANES)
            off_v[sl] = idx_v[sl] + base
        pltpu.async_copy(flat_hbm.at[off_v], out_v, sem).wait()
        pltpu.sync_copy(out_v, out_hbm.at[pl.ds(d * b, b)])

    return gather_kernel(flat_table, idx)


def _tc_project_t(wt, et_flat, d, b, vt):
    v = wt.shape[1]

    def body(wt_ref, et_ref, o_ref):
        et = et_ref[...].reshape(d, b)
        o_ref[...] = lax.dot_general(
            wt_ref[...],
            et,
            (((0,), (0,)), ((), ())),
            preferred_element_type=jnp.float32,
        )

    return pl.pallas_call(
        body,
        grid=(pl.cdiv(v, vt),),
        in_specs=[
            pl.BlockSpec((d, vt), lambda j: (0, j)),
            pl.BlockSpec((d * b,), lambda j: (0,)),
        ],
        out_specs=pl.BlockSpec((vt, b), lambda j: (j, 0)),
        out_shape=jax.ShapeDtypeStruct((v, b), jnp.float32),
    )(wt, et_flat)


def kernel(x, emb_table, out_weight):
    rows, d = emb_table.shape
    b = x.shape[0]
    flat_t = emb_table.T.reshape(-1)
    et_flat = _sc_gather_t(flat_t, x.astype(jnp.int32), d, rows)
    logits_t = _tc_project_t(out_weight.T, et_flat, d, b, vt=4096)
    return logits_t.T
